# baseline (speedup 1.0000x reference)
import dataclasses
import functools

import jax
import jax.numpy as jnp
from jax import lax
from jax.experimental import pallas as pl
from jax.experimental.pallas import tpu as pltpu
from jax.experimental.pallas import tpu_sc as plsc

P, NBR, D, H = 8192, 16, 512, 16
HD = D // H
G = H * 3
GP = 128
R = P * NBR

_SC_CP = pltpu.CompilerParams()
if "needs_layout_passes" in pltpu.CompilerParams.__dataclass_fields__:
    _SC_CP = dataclasses.replace(_SC_CP, needs_layout_passes=False)
NH = NBR * H
NV = D // 16


BLK1 = 256


def _bits16(x):
    b = jax.lax.bitcast_convert_type(x.astype(jnp.bfloat16), jnp.uint16)
    return b.astype(jnp.uint32)


def _proj_kernel(xq_ref, xk_ref, xv_ref, wq_ref, bq_ref, wke_ref, bke_ref,
                 wko_ref, bko_ref, wve_ref, bve_ref, wvo_ref, bvo_ref,
                 wg1q_ref, wg1e_ref, wg1o_ref, bg1_ref,
                 q_ref, k_ref, v_ref, a_ref, bqo_ref):
    f32 = jnp.float32
    q = jnp.dot(xq_ref[...], wq_ref[...], preferred_element_type=f32) + bq_ref[...]
    ke = jnp.dot(xk_ref[...], wke_ref[...], preferred_element_type=f32) + bke_ref[...]
    ko = jnp.dot(xk_ref[...], wko_ref[...], preferred_element_type=f32) + bko_ref[...]
    ve = jnp.dot(xv_ref[...], wve_ref[...], preferred_element_type=f32) + bve_ref[...]
    vo = jnp.dot(xv_ref[...], wvo_ref[...], preferred_element_type=f32) + bvo_ref[...]
    q_ref[...] = q
    k_ref[...] = jax.lax.bitcast_convert_type(
        _bits16(ke) | (_bits16(ko) << 16), jnp.int32)
    v_ref[...] = jax.lax.bitcast_convert_type(
        _bits16(ve) | (_bits16(vo) << 16), jnp.int32)
    a_ref[...] = (jnp.dot(ke, wg1e_ref[...], preferred_element_type=f32)
                  + jnp.dot(ko, wg1o_ref[...], preferred_element_type=f32))
    bqo_ref[...] = jnp.dot(q, wg1q_ref[...], preferred_element_type=f32) - bg1_ref[...]


def _stage1(query, key, value, Wqh, bqh, Wke, bke, Wko, bko, Wve, bve,
            Wvo, bvo, Wg1q, Wg1e, Wg1o, bg1):
    n_blk = P // BLK1
    DH = D // 2
    row_spec = pl.BlockSpec((BLK1, D), lambda i: (i, 0))
    w_spec = pl.BlockSpec((D, D), lambda i: (0, 0))
    b_spec = pl.BlockSpec((1, D), lambda i: (0, 0))
    wh_spec = pl.BlockSpec((D, DH), lambda i: (0, 0))
    bh_spec = pl.BlockSpec((1, DH), lambda i: (0, 0))
    g_spec = pl.BlockSpec((D, GP), lambda i: (0, 0))
    gh_spec = pl.BlockSpec((DH, GP), lambda i: (0, 0))
    gb_spec = pl.BlockSpec((1, GP), lambda i: (0, 0))
    out_row = pl.BlockSpec((BLK1, D), lambda i: (i, 0))
    out_half = pl.BlockSpec((BLK1, DH), lambda i: (i, 0))
    out_g = pl.BlockSpec((BLK1, GP), lambda i: (i, 0))
    return pl.pallas_call(
        _proj_kernel,
        grid=(n_blk,),
        in_specs=[row_spec, row_spec, row_spec, w_spec, b_spec,
                  wh_spec, bh_spec, wh_spec, bh_spec,
                  wh_spec, bh_spec, wh_spec, bh_spec,
                  g_spec, gh_spec, gh_spec, gb_spec],
        out_specs=[out_row, out_half, out_half, out_g, out_g],
        out_shape=[
            jax.ShapeDtypeStruct((P, D), jnp.float32),
            jax.ShapeDtypeStruct((P, DH), jnp.int32),
            jax.ShapeDtypeStruct((P, DH), jnp.int32),
            jax.ShapeDtypeStruct((P, GP), jnp.float32),
            jax.ShapeDtypeStruct((P, GP), jnp.float32),
        ],
    )(query, key, value, Wqh, bqh.reshape(1, D), Wke, bke.reshape(1, DH),
      Wko, bko.reshape(1, DH), Wve, bve.reshape(1, DH), Wvo,
      bvo.reshape(1, DH), Wg1q, Wg1e, Wg1o, bg1.reshape(1, GP))


NW = 32
PH = P // 2
QW = PH // NW
QB = 4
NCH = QW // QB


def _make_sc_dots(off):
  def _sc_dots(k_hbm, a_hbm, q_hbm, idx_hbm, dots_hbm, ag_hbm,
               idxall, kbuf0, kbuf1, abuf0, abuf1, qbuf0, qbuf1, dotbuf,
               semk0, semk1, sema0, sema1, semq0, semq1):
    wid = lax.axis_index("s") * 2 + lax.axis_index("c")
    q0l = wid * QW
    q0 = off + q0l
    kbuf = [kbuf0, kbuf1]
    abuf = [abuf0, abuf1]
    qbuf = [qbuf0, qbuf1]
    semk = [semk0, semk1]
    sema = [sema0, sema1]
    semq = [semq0, semq1]

    pltpu.sync_copy(idx_hbm.at[pl.ds(q0 * NBR, QW * NBR)], idxall)

    def prefetch(ch, b):
        qb = q0 + ch * QB
        idxs = idxall.at[pl.ds(ch * QB * NBR, QB * NBR)]
        pltpu.async_copy(q_hbm.at[pl.ds(qb, QB)], qbuf[b], semq[b])
        pltpu.async_copy(k_hbm.at[idxs], kbuf[b], semk[b])
        pltpu.async_copy(a_hbm.at[idxs], abuf[b], sema[b])

    def process(ch, b):
        qb = q0 + ch * QB
        pb = (q0l + ch * QB) * NBR
        idxs = idxall.at[pl.ds(ch * QB * NBR, QB * NBR)]
        pltpu.make_async_copy(k_hbm.at[idxs], kbuf[b], semk[b]).wait()
        pltpu.make_async_copy(a_hbm.at[idxs], abuf[b], sema[b]).wait()
        pltpu.make_async_copy(q_hbm.at[pl.ds(qb, QB)], qbuf[b], semq[b]).wait()

        @pl.loop(0, QB)
        def _(jq):
            qv = [qbuf[b][jq, pl.ds(c * 16, 16)] for c in range(NV)]
            for n in range(NBR):
                row = jq * NBR + n
                acc = None
                for m in range(NV // 2):
                    kk = plsc.bitcast(kbuf[b][row, pl.ds(m * 16, 16)],
                                      jnp.bfloat16)
                    ka, kb = plsc.unpack(
                        kk, format=plsc.PackFormat.INTERLEAVED)
                    term = qv[2 * m] * ka + qv[2 * m + 1] * kb
                    acc = term if acc is None else acc + term
                dotbuf[row, pl.ds(0, 16)] = acc

        pltpu.sync_copy(dotbuf, dots_hbm.at[pl.ds(pb, QB * NBR)])
        pltpu.sync_copy(abuf[b], ag_hbm.at[pl.ds(pb, QB * NBR)])

    prefetch(0, 0)

    @pl.loop(0, NCH // 2)
    def _(t):
        ch0 = t * 2
        prefetch(ch0 + 1, 1)
        process(ch0, 0)
        ch2 = ch0 + 2

        @pl.when(ch2 < NCH)
        def _():
            prefetch(ch2, 0)

        process(ch0 + 1, 1)

  return _sc_dots


def _sc_pass1(k, a, q, idx_flat, off):
    mesh = plsc.VectorSubcoreMesh(core_axis_name="c", subcore_axis_name="s")
    kern = functools.partial(
        pl.kernel,
        out_type=(
            jax.ShapeDtypeStruct((PH * NBR, H), jnp.float32),
            jax.ShapeDtypeStruct((PH * NBR, GP), jnp.float32),
        ),
        mesh=mesh,
        compiler_params=_SC_CP,
        scratch_types=[
            pltpu.VMEM((QW * NBR,), jnp.int32),
            pltpu.VMEM((QB * NBR, D // 2), jnp.int32),
            pltpu.VMEM((QB * NBR, D // 2), jnp.int32),
            pltpu.VMEM((QB * NBR, GP), jnp.float32),
            pltpu.VMEM((QB * NBR, GP), jnp.float32),
            pltpu.VMEM((QB, D), jnp.float32),
            pltpu.VMEM((QB, D), jnp.float32),
            pltpu.VMEM((QB * NBR, H), jnp.float32),
            pltpu.SemaphoreType.DMA,
            pltpu.SemaphoreType.DMA,
            pltpu.SemaphoreType.DMA,
            pltpu.SemaphoreType.DMA,
            pltpu.SemaphoreType.DMA,
            pltpu.SemaphoreType.DMA,
        ],
    )(_make_sc_dots(off))
    return kern(k, a, q, idx_flat)


SROWS = R // 128


def _scal_kernel(x_ref, s3_ref, s_ref):
    x = x_ref[...]
    n2 = jnp.dot(x * x, s3_ref[...], preferred_element_type=jnp.float32,
                 precision=lax.Precision.HIGHEST)
    ln = jnp.sqrt(n2)
    a_c, b_c = 0.001, 0.005
    ramp = 0.5 * (1.0 - jnp.cos(jnp.pi * (ln - a_c) / (b_c - a_c)))
    cut = jnp.where(ln < a_c, 0.0, jnp.where(ln > b_c, 1.0, ramp))
    s_ref[...] = jnp.sqrt(3.0) * cut / jnp.maximum(ln, 1e-12)


def _stage_s(rpe_pack, S3):
    return pl.pallas_call(
        _scal_kernel,
        grid=(1,),
        in_specs=[pl.BlockSpec((SROWS, 384), lambda i: (0, 0)),
                  pl.BlockSpec((384, 128), lambda i: (0, 0))],
        out_specs=pl.BlockSpec((SROWS, 128), lambda i: (0, 0)),
        out_shape=jax.ShapeDtypeStruct((SROWS, 128), jnp.float32),
    )(rpe_pack, S3)


BLKM = 128
RBM = BLKM * NBR


def _mid_kernel(dot_ref, bq_ref, ag_ref, rpe_ref, s_ref, rel_ref,
                wg2_ref, bg2_ref, m48_ref, t3_ref, att_ref):
    bq = bq_ref[...]
    bqexp = jnp.broadcast_to(bq[:, None, :], (BLKM, NBR, GP)).reshape(RBM, GP)
    pre = jnp.maximum(ag_ref[...] - bqexp, 0.0)
    t = jnp.dot(pre, wg2_ref[...], preferred_element_type=jnp.float32) + bg2_ref[...]

    shc = rpe_ref[...].reshape(RBM, 3) * s_ref[...]
    sht = jnp.dot(shc, t3_ref[...], preferred_element_type=jnp.float32)

    pos = jnp.dot(t * sht, m48_ref[...], preferred_element_type=jnp.float32)

    dot = dot_ref[...]
    lg = ((dot + pos).reshape(BLKM, NBR, H) + rel_ref[...]) * (
        1.0 / jnp.sqrt(jnp.float32(HD)))
    m = jnp.max(lg, axis=1, keepdims=True)
    e = jnp.exp(lg - m)
    w = e / jnp.sum(e, axis=1, keepdims=True)
    att_ref[...] = w.reshape(RBM, H)


def _stage_mid(dots, bqv, ag, rpe3, s_flat, rel3, Wg2, bg2, M48, T3):
    n_blk = dots.shape[0] // RBM
    return pl.pallas_call(
        _mid_kernel,
        grid=(n_blk,),
        in_specs=[
            pl.BlockSpec((RBM, H), lambda i: (i, 0)),
            pl.BlockSpec((BLKM, GP), lambda i: (i, 0)),
            pl.BlockSpec((RBM, GP), lambda i: (i, 0)),
            pl.BlockSpec((BLKM, NBR, 3), lambda i: (i, 0, 0)),
            pl.BlockSpec((RBM, 1), lambda i: (i, 0)),
            pl.BlockSpec((BLKM, NBR, H), lambda i: (i, 0, 0)),
            pl.BlockSpec((GP, GP), lambda i: (0, 0)),
            pl.BlockSpec((1, GP), lambda i: (0, 0)),
            pl.BlockSpec((GP, H), lambda i: (0, 0)),
            pl.BlockSpec((3, GP), lambda i: (0, 0)),
        ],
        out_specs=pl.BlockSpec((RBM, H), lambda i: (i, 0)),
        out_shape=jax.ShapeDtypeStruct((dots.shape[0], H), jnp.float32),
    )(dots, bqv, ag, rpe3, s_flat, rel3, Wg2, bg2.reshape(1, GP),
      M48, T3)


def _make_sc_wsum(off):
  def _sc_wsum(v_hbm, att_hbm, idx_hbm, out_hbm,
               idxall, vbuf0, vbuf1, attbuf0, attbuf1, outbuf,
               semv0, semv1, sematt0, sematt1):
    wid = lax.axis_index("s") * 2 + lax.axis_index("c")
    q0l = wid * QW
    q0 = off + q0l
    vbuf = [vbuf0, vbuf1]
    attbuf = [attbuf0, attbuf1]
    semv = [semv0, semv1]
    sematt = [sematt0, sematt1]

    pltpu.sync_copy(idx_hbm.at[pl.ds(q0 * NBR, QW * NBR)], idxall)


    def prefetch(ch, b):
        pb = (q0l + ch * QB) * NBR
        idxs = idxall.at[pl.ds(ch * QB * NBR, QB * NBR)]
        pltpu.async_copy(att_hbm.at[pl.ds(pb, QB * NBR)], attbuf[b], sematt[b])
        pltpu.async_copy(v_hbm.at[idxs], vbuf[b], semv[b])

    def process(ch, b):
        qb = q0l + ch * QB
        pb = qb * NBR
        idxs = idxall.at[pl.ds(ch * QB * NBR, QB * NBR)]
        pltpu.make_async_copy(v_hbm.at[idxs], vbuf[b], semv[b]).wait()
        pltpu.make_async_copy(att_hbm.at[pl.ds(pb, QB * NBR)], attbuf[b],
                              sematt[b]).wait()

        @pl.loop(0, QB)
        def _(jq):
            att = [attbuf[b][jq * NBR + n, pl.ds(0, 16)] for n in range(NBR)]
            for m in range(NV // 2):
                acc_a = None
                acc_b = None
                for n in range(NBR):
                    vv = plsc.bitcast(
                        vbuf[b][jq * NBR + n, pl.ds(m * 16, 16)],
                        jnp.bfloat16)
                    va, vb = plsc.unpack(
                        vv, format=plsc.PackFormat.INTERLEAVED)
                    ta = att[n] * va
                    tb = att[n] * vb
                    acc_a = ta if acc_a is None else acc_a + ta
                    acc_b = tb if acc_b is None else acc_b + tb
                outbuf[jq, pl.ds((2 * m) * 16, 16)] = acc_a
                outbuf[jq, pl.ds((2 * m + 1) * 16, 16)] = acc_b

        pltpu.sync_copy(outbuf, out_hbm.at[pl.ds(qb, QB)])

    prefetch(0, 0)

    @pl.loop(0, NCH // 2)
    def _(t):
        ch0 = t * 2
        prefetch(ch0 + 1, 1)
        process(ch0, 0)
        ch2 = ch0 + 2

        @pl.when(ch2 < NCH)
        def _():
            prefetch(ch2, 0)

        process(ch0 + 1, 1)

  return _sc_wsum


def _sc_pass2(v, att, idx_flat, off):
    mesh = plsc.VectorSubcoreMesh(core_axis_name="c", subcore_axis_name="s")
    kern = functools.partial(
        pl.kernel,
        out_type=jax.ShapeDtypeStruct((PH, D), jnp.float32),
        mesh=mesh,
        compiler_params=_SC_CP,
        scratch_types=[
            pltpu.VMEM((QW * NBR,), jnp.int32),
            pltpu.VMEM((QB * NBR, D // 2), jnp.int32),
            pltpu.VMEM((QB * NBR, D // 2), jnp.int32),
            pltpu.VMEM((QB * NBR, H), jnp.float32),
            pltpu.VMEM((QB * NBR, H), jnp.float32),
            pltpu.VMEM((QB, D), jnp.float32),
            pltpu.SemaphoreType.DMA,
            pltpu.SemaphoreType.DMA,
            pltpu.SemaphoreType.DMA,
            pltpu.SemaphoreType.DMA,
        ],
    )(_make_sc_wsum(off))
    return kern(v, att, idx_flat)


BLKU = 512


def _unperm_kernel(x_ref, pm_ref, o_ref):
    o_ref[...] = jnp.dot(x_ref[...], pm_ref[...],
                         preferred_element_type=jnp.float32,
                         precision=lax.Precision.HIGHEST)


def _stage_unperm(x, Pm):
    return pl.pallas_call(
        _unperm_kernel,
        grid=(P // BLKU,),
        in_specs=[pl.BlockSpec((BLKU, D), lambda i: (i, 0)),
                  pl.BlockSpec((D, D), lambda i: (0, 0))],
        out_specs=pl.BlockSpec((BLKU, D), lambda i: (i, 0)),
        out_shape=jax.ShapeDtypeStruct((P, D), jnp.float32),
    )(x, Pm)


def _consts():
    h = jnp.arange(H)
    j = jnp.arange(G)
    M48 = (j[:, None] // 3 == h[None, :]).astype(jnp.float32)
    M48 = jnp.pad(M48, ((0, GP - G), (0, 0)))
    permc = jnp.array([1, 2, 0])
    c3 = jnp.arange(3)
    T3 = (c3[:, None] == permc[j % 3][None, :]).astype(jnp.float32)
    T3 = jnp.pad(T3, ((0, 0), (0, GP - G)))
    j384 = jnp.arange(384)
    S3 = (j384[:, None] // 3 == jnp.arange(128)[None, :]).astype(jnp.float32)
    c = jnp.arange(D)
    perm_hat = (c % 16) * HD + c // 16
    m2 = c // 32
    r2 = c % 32
    perm_hat2 = (r2 // 2) * HD + 2 * m2 + (r2 % 2)
    d = jnp.arange(D)
    hatpos = (d % HD) * H + d // HD
    Pm = (hatpos[:, None] == jnp.arange(D)[None, :]).astype(jnp.float32).T
    return M48, T3, S3, perm_hat, perm_hat2, Pm


@jax.jit
def kernel(query, key, value, index_pair, query_batch_cnt, key_batch_cnt,
           index_pair_batch, relative_atten_weights, rpe_distance,
           Wq, bq, Wk, bk, Wv, bv, Wg1, bg1, Wg2, bg2):
    M48, T3, S3, perm_hat, perm_hat2, Pm = _consts()
    Wqh = Wq[:, perm_hat]
    bqh = bq[perm_hat]
    pe = perm_hat2[0::2]
    po = perm_hat2[1::2]
    Wke, Wko = Wk[:, pe], Wk[:, po]
    bke, bko = bk[pe], bk[po]
    Wve, Wvo = Wv[:, pe], Wv[:, po]
    bve, bvo = bv[pe], bv[po]
    Wg1pad = jnp.pad(Wg1, ((0, 0), (0, GP - G)))
    Wg1h = Wg1pad[perm_hat, :]
    Wg1e = Wg1pad[pe, :]
    Wg1o = Wg1pad[po, :]
    bg1p = jnp.pad(bg1, (0, GP - G))
    Wg2p = jnp.pad(Wg2, ((0, GP - G), (0, GP - G)))
    bg2p = jnp.pad(bg2, (0, GP - G))

    q, kp, vp, a, bqv = _stage1(query, key, value, Wqh, bqh, Wke, bke,
                                Wko, bko, Wve, bve, Wvo, bvo,
                                Wg1h, Wg1e, Wg1o, bg1p)
    idx_flat = index_pair.reshape(R)
    s_pack = _stage_s(rpe_distance.reshape(SROWS, 384), S3)
    s_flat = s_pack.reshape(R, 1)
    outs = []
    for off in (0, PH):
        dots_h, ag_h = _sc_pass1(kp, a, q, idx_flat, off)
        att_h = _stage_mid(dots_h, bqv[off:off + PH],
                           ag_h, rpe_distance[off:off + PH],
                           s_flat[off * NBR:(off + PH) * NBR],
                           relative_atten_weights[off:off + PH],
                           Wg2p, bg2p, M48, T3)
        outs.append(_sc_pass2(vp, att_h, idx_flat, off))
    out_hat = jnp.concatenate(outs, axis=0)
    return _stage_unperm(out_hat, Pm)


# speedup vs baseline: 1.0747x; 1.0747x over previous
import dataclasses
import functools

import jax
import jax.numpy as jnp
from jax import lax
from jax.experimental import pallas as pl
from jax.experimental.pallas import tpu as pltpu
from jax.experimental.pallas import tpu_sc as plsc

P, NBR, D, H = 8192, 16, 512, 16
HD = D // H
G = H * 3
GP = 128
R = P * NBR

_SC_CP = pltpu.CompilerParams()
if "needs_layout_passes" in pltpu.CompilerParams.__dataclass_fields__:
    _SC_CP = dataclasses.replace(_SC_CP, needs_layout_passes=False)
NH = NBR * H
NV = D // 16


BLK1 = 256


def _bits16(x):
    b = jax.lax.bitcast_convert_type(x.astype(jnp.bfloat16), jnp.uint16)
    return b.astype(jnp.uint32)


def _proj_kernel(xq_ref, xk_ref, xv_ref, wq_ref, bq_ref, wke_ref, bke_ref,
                 wko_ref, bko_ref, wve_ref, bve_ref, wvo_ref, bvo_ref,
                 wg1q_ref, wg1e_ref, wg1o_ref, bg1_ref,
                 q_ref, k_ref, v_ref, a_ref, bqo_ref):
    f32 = jnp.float32
    q = jnp.dot(xq_ref[...], wq_ref[...], preferred_element_type=f32) + bq_ref[...]
    ke = jnp.dot(xk_ref[...], wke_ref[...], preferred_element_type=f32) + bke_ref[...]
    ko = jnp.dot(xk_ref[...], wko_ref[...], preferred_element_type=f32) + bko_ref[...]
    ve = jnp.dot(xv_ref[...], wve_ref[...], preferred_element_type=f32) + bve_ref[...]
    vo = jnp.dot(xv_ref[...], wvo_ref[...], preferred_element_type=f32) + bvo_ref[...]
    q_ref[...] = q
    k_ref[...] = jax.lax.bitcast_convert_type(
        _bits16(ke) | (_bits16(ko) << 16), jnp.int32)
    v_ref[...] = jax.lax.bitcast_convert_type(
        _bits16(ve) | (_bits16(vo) << 16), jnp.int32)
    a_ref[...] = (jnp.dot(ke, wg1e_ref[...], preferred_element_type=f32)
                  + jnp.dot(ko, wg1o_ref[...], preferred_element_type=f32))
    bqo_ref[...] = jnp.dot(q, wg1q_ref[...], preferred_element_type=f32) - bg1_ref[...]


def _stage1(query, key, value, Wqh, bqh, Wke, bke, Wko, bko, Wve, bve,
            Wvo, bvo, Wg1q, Wg1e, Wg1o, bg1):
    n_blk = P // BLK1
    DH = D // 2
    row_spec = pl.BlockSpec((BLK1, D), lambda i: (i, 0))
    w_spec = pl.BlockSpec((D, D), lambda i: (0, 0))
    b_spec = pl.BlockSpec((1, D), lambda i: (0, 0))
    wh_spec = pl.BlockSpec((D, DH), lambda i: (0, 0))
    bh_spec = pl.BlockSpec((1, DH), lambda i: (0, 0))
    g_spec = pl.BlockSpec((D, GP), lambda i: (0, 0))
    gh_spec = pl.BlockSpec((DH, GP), lambda i: (0, 0))
    gb_spec = pl.BlockSpec((1, GP), lambda i: (0, 0))
    out_row = pl.BlockSpec((BLK1, D), lambda i: (i, 0))
    out_half = pl.BlockSpec((BLK1, DH), lambda i: (i, 0))
    out_g = pl.BlockSpec((BLK1, GP), lambda i: (i, 0))
    return pl.pallas_call(
        _proj_kernel,
        grid=(n_blk,),
        in_specs=[row_spec, row_spec, row_spec, w_spec, b_spec,
                  wh_spec, bh_spec, wh_spec, bh_spec,
                  wh_spec, bh_spec, wh_spec, bh_spec,
                  g_spec, gh_spec, gh_spec, gb_spec],
        out_specs=[out_row, out_half, out_half, out_g, out_g],
        out_shape=[
            jax.ShapeDtypeStruct((P, D), jnp.float32),
            jax.ShapeDtypeStruct((P, DH), jnp.int32),
            jax.ShapeDtypeStruct((P, DH), jnp.int32),
            jax.ShapeDtypeStruct((P, GP), jnp.float32),
            jax.ShapeDtypeStruct((P, GP), jnp.float32),
        ],
    )(query, key, value, Wqh, bqh.reshape(1, D), Wke, bke.reshape(1, DH),
      Wko, bko.reshape(1, DH), Wve, bve.reshape(1, DH), Wvo,
      bvo.reshape(1, DH), Wg1q, Wg1e, Wg1o, bg1.reshape(1, GP))


NW = 32
PH = P // 2
QW = PH // NW
QB = 4
NCH = QW // QB


def _make_sc_dots(off):
  def _sc_dots(k_hbm, a_hbm, q_hbm, idx_hbm, dots_hbm, ag_hbm,
               idxall, kbuf0, kbuf1, abuf0, abuf1, qbuf0, qbuf1, dotbuf,
               semk0, semk1, sema0, sema1, semq0, semq1):
    wid = lax.axis_index("s") * 2 + lax.axis_index("c")
    q0l = wid * QW
    q0 = off + q0l
    kbuf = [kbuf0, kbuf1]
    abuf = [abuf0, abuf1]
    qbuf = [qbuf0, qbuf1]
    semk = [semk0, semk1]
    sema = [sema0, sema1]
    semq = [semq0, semq1]

    pltpu.sync_copy(idx_hbm.at[pl.ds(q0 * NBR, QW * NBR)], idxall)

    def prefetch(ch, b):
        qb = q0 + ch * QB
        idxs = idxall.at[pl.ds(ch * QB * NBR, QB * NBR)]
        pltpu.async_copy(q_hbm.at[pl.ds(qb, QB)], qbuf[b], semq[b])
        pltpu.async_copy(k_hbm.at[idxs], kbuf[b], semk[b])
        pltpu.async_copy(a_hbm.at[idxs], abuf[b], sema[b])

    def process(ch, b):
        qb = q0 + ch * QB
        pb = (q0l + ch * QB) * NBR
        idxs = idxall.at[pl.ds(ch * QB * NBR, QB * NBR)]
        pltpu.make_async_copy(k_hbm.at[idxs], kbuf[b], semk[b]).wait()
        pltpu.make_async_copy(a_hbm.at[idxs], abuf[b], sema[b]).wait()
        pltpu.make_async_copy(q_hbm.at[pl.ds(qb, QB)], qbuf[b], semq[b]).wait()

        @pl.loop(0, QB)
        def _(jq):
            qv = [qbuf[b][jq, pl.ds(c * 16, 16)] for c in range(NV)]
            for n in range(NBR):
                row = jq * NBR + n
                acc = None
                for m in range(NV // 2):
                    kk = plsc.bitcast(kbuf[b][row, pl.ds(m * 16, 16)],
                                      jnp.bfloat16)
                    ka, kb = plsc.unpack(
                        kk, format=plsc.PackFormat.INTERLEAVED)
                    term = qv[2 * m] * ka + qv[2 * m + 1] * kb
                    acc = term if acc is None else acc + term
                dotbuf[row, pl.ds(0, 16)] = acc

        pltpu.sync_copy(dotbuf, dots_hbm.at[pl.ds(pb, QB * NBR)])
        pltpu.sync_copy(abuf[b], ag_hbm.at[pl.ds(pb, QB * NBR)])

    prefetch(0, 0)

    @pl.loop(0, NCH // 2)
    def _(t):
        ch0 = t * 2
        prefetch(ch0 + 1, 1)
        process(ch0, 0)
        ch2 = ch0 + 2

        @pl.when(ch2 < NCH)
        def _():
            prefetch(ch2, 0)

        process(ch0 + 1, 1)

  return _sc_dots


def _sc_pass1(k, a, q, idx_flat, off):
    mesh = plsc.VectorSubcoreMesh(core_axis_name="c", subcore_axis_name="s")
    kern = functools.partial(
        pl.kernel,
        out_type=(
            jax.ShapeDtypeStruct((PH * NBR, H), jnp.float32),
            jax.ShapeDtypeStruct((PH * NBR, GP), jnp.float32),
        ),
        mesh=mesh,
        compiler_params=_SC_CP,
        scratch_types=[
            pltpu.VMEM((QW * NBR,), jnp.int32),
            pltpu.VMEM((QB * NBR, D // 2), jnp.int32),
            pltpu.VMEM((QB * NBR, D // 2), jnp.int32),
            pltpu.VMEM((QB * NBR, GP), jnp.float32),
            pltpu.VMEM((QB * NBR, GP), jnp.float32),
            pltpu.VMEM((QB, D), jnp.float32),
            pltpu.VMEM((QB, D), jnp.float32),
            pltpu.VMEM((QB * NBR, H), jnp.float32),
            pltpu.SemaphoreType.DMA,
            pltpu.SemaphoreType.DMA,
            pltpu.SemaphoreType.DMA,
            pltpu.SemaphoreType.DMA,
            pltpu.SemaphoreType.DMA,
            pltpu.SemaphoreType.DMA,
        ],
    )(_make_sc_dots(off))
    return kern(k, a, q, idx_flat)


SROWS = R // 128


def _scal_kernel(x_ref, s3_ref, s_ref):
    x = x_ref[...]
    n2 = jnp.dot(x * x, s3_ref[...], preferred_element_type=jnp.float32,
                 precision=lax.Precision.HIGHEST)
    ln = jnp.sqrt(n2)
    a_c, b_c = 0.001, 0.005
    ramp = 0.5 * (1.0 - jnp.cos(jnp.pi * (ln - a_c) / (b_c - a_c)))
    cut = jnp.where(ln < a_c, 0.0, jnp.where(ln > b_c, 1.0, ramp))
    s_ref[...] = jnp.sqrt(3.0) * cut / jnp.maximum(ln, 1e-12)


def _stage_s(rpe_pack, S3):
    return pl.pallas_call(
        _scal_kernel,
        grid=(1,),
        in_specs=[pl.BlockSpec((SROWS, 384), lambda i: (0, 0)),
                  pl.BlockSpec((384, 128), lambda i: (0, 0))],
        out_specs=pl.BlockSpec((SROWS, 128), lambda i: (0, 0)),
        out_shape=jax.ShapeDtypeStruct((SROWS, 128), jnp.float32),
    )(rpe_pack, S3)


BLKM = 128
RBM = BLKM * NBR


def _mid_kernel(dot_ref, bq_ref, ag_ref, rpe_ref, s_ref, rel_ref,
                wg2_ref, bg2_ref, m48_ref, t3_ref, att_ref):
    bq = bq_ref[...]
    bqexp = jnp.broadcast_to(bq[:, None, :], (BLKM, NBR, GP)).reshape(RBM, GP)
    pre = jnp.maximum(ag_ref[...] - bqexp, 0.0)
    t = jnp.dot(pre, wg2_ref[...], preferred_element_type=jnp.float32) + bg2_ref[...]

    shc = rpe_ref[...].reshape(RBM, 3) * s_ref[...]
    sht = jnp.dot(shc, t3_ref[...], preferred_element_type=jnp.float32)

    pos = jnp.dot(t * sht, m48_ref[...], preferred_element_type=jnp.float32)

    dot = dot_ref[...]
    lg = ((dot + pos).reshape(BLKM, NBR, H) + rel_ref[...]) * (
        1.0 / jnp.sqrt(jnp.float32(HD)))
    m = jnp.max(lg, axis=1, keepdims=True)
    e = jnp.exp(lg - m)
    w = e / jnp.sum(e, axis=1, keepdims=True)
    att_ref[...] = w.reshape(RBM, H)


def _stage_mid(dots, bqv, ag, rpe3, s_flat, rel3, Wg2, bg2, M48, T3, off):
    n_blk = dots.shape[0] // RBM
    sb = off // BLKM
    sp = off * NBR // RBM
    return pl.pallas_call(
        _mid_kernel,
        grid=(n_blk,),
        in_specs=[
            pl.BlockSpec((RBM, H), lambda i: (i, 0)),
            pl.BlockSpec((BLKM, GP), lambda i: (i + sb, 0)),
            pl.BlockSpec((RBM, GP), lambda i: (i, 0)),
            pl.BlockSpec((BLKM, NBR, 3), lambda i: (i + sb, 0, 0)),
            pl.BlockSpec((RBM, 1), lambda i: (i + sp, 0)),
            pl.BlockSpec((BLKM, NBR, H), lambda i: (i + sb, 0, 0)),
            pl.BlockSpec((GP, GP), lambda i: (0, 0)),
            pl.BlockSpec((1, GP), lambda i: (0, 0)),
            pl.BlockSpec((GP, H), lambda i: (0, 0)),
            pl.BlockSpec((3, GP), lambda i: (0, 0)),
        ],
        out_specs=pl.BlockSpec((RBM, H), lambda i: (i, 0)),
        out_shape=jax.ShapeDtypeStruct((dots.shape[0], H), jnp.float32),
    )(dots, bqv, ag, rpe3, s_flat, rel3, Wg2, bg2.reshape(1, GP),
      M48, T3)


def _make_sc_wsum(off):
  def _sc_wsum(v_hbm, att_hbm, idx_hbm, out_hbm,
               idxall, vbuf0, vbuf1, attbuf0, attbuf1, outbuf,
               semv0, semv1, sematt0, sematt1):
    wid = lax.axis_index("s") * 2 + lax.axis_index("c")
    q0l = wid * QW
    q0 = off + q0l
    vbuf = [vbuf0, vbuf1]
    attbuf = [attbuf0, attbuf1]
    semv = [semv0, semv1]
    sematt = [sematt0, sematt1]

    pltpu.sync_copy(idx_hbm.at[pl.ds(q0 * NBR, QW * NBR)], idxall)


    def prefetch(ch, b):
        pb = (q0l + ch * QB) * NBR
        idxs = idxall.at[pl.ds(ch * QB * NBR, QB * NBR)]
        pltpu.async_copy(att_hbm.at[pl.ds(pb, QB * NBR)], attbuf[b], sematt[b])
        pltpu.async_copy(v_hbm.at[idxs], vbuf[b], semv[b])

    def process(ch, b):
        qb = q0l + ch * QB
        pb = qb * NBR
        idxs = idxall.at[pl.ds(ch * QB * NBR, QB * NBR)]
        pltpu.make_async_copy(v_hbm.at[idxs], vbuf[b], semv[b]).wait()
        pltpu.make_async_copy(att_hbm.at[pl.ds(pb, QB * NBR)], attbuf[b],
                              sematt[b]).wait()

        @pl.loop(0, QB)
        def _(jq):
            att = [attbuf[b][jq * NBR + n, pl.ds(0, 16)] for n in range(NBR)]
            for m in range(NV // 2):
                acc_a = None
                acc_b = None
                for n in range(NBR):
                    vv = plsc.bitcast(
                        vbuf[b][jq * NBR + n, pl.ds(m * 16, 16)],
                        jnp.bfloat16)
                    va, vb = plsc.unpack(
                        vv, format=plsc.PackFormat.INTERLEAVED)
                    ta = att[n] * va
                    tb = att[n] * vb
                    acc_a = ta if acc_a is None else acc_a + ta
                    acc_b = tb if acc_b is None else acc_b + tb
                outbuf[jq, pl.ds((2 * m) * 16, 16)] = acc_a
                outbuf[jq, pl.ds((2 * m + 1) * 16, 16)] = acc_b

        pltpu.sync_copy(outbuf, out_hbm.at[pl.ds(qb, QB)])

    prefetch(0, 0)

    @pl.loop(0, NCH // 2)
    def _(t):
        ch0 = t * 2
        prefetch(ch0 + 1, 1)
        process(ch0, 0)
        ch2 = ch0 + 2

        @pl.when(ch2 < NCH)
        def _():
            prefetch(ch2, 0)

        process(ch0 + 1, 1)

  return _sc_wsum


def _sc_pass2(v, att, idx_flat, off):
    mesh = plsc.VectorSubcoreMesh(core_axis_name="c", subcore_axis_name="s")
    kern = functools.partial(
        pl.kernel,
        out_type=jax.ShapeDtypeStruct((PH, D), jnp.float32),
        mesh=mesh,
        compiler_params=_SC_CP,
        scratch_types=[
            pltpu.VMEM((QW * NBR,), jnp.int32),
            pltpu.VMEM((QB * NBR, D // 2), jnp.int32),
            pltpu.VMEM((QB * NBR, D // 2), jnp.int32),
            pltpu.VMEM((QB * NBR, H), jnp.float32),
            pltpu.VMEM((QB * NBR, H), jnp.float32),
            pltpu.VMEM((QB, D), jnp.float32),
            pltpu.SemaphoreType.DMA,
            pltpu.SemaphoreType.DMA,
            pltpu.SemaphoreType.DMA,
            pltpu.SemaphoreType.DMA,
        ],
    )(_make_sc_wsum(off))
    return kern(v, att, idx_flat)


BLKU = 512


def _unperm_kernel(x_ref, pm_ref, o_ref):
    o_ref[...] = jnp.dot(x_ref[...], pm_ref[...],
                         preferred_element_type=jnp.float32,
                         precision=lax.Precision.HIGHEST)


def _stage_unperm(x, Pm):
    return pl.pallas_call(
        _unperm_kernel,
        grid=(P // BLKU,),
        in_specs=[pl.BlockSpec((BLKU, D), lambda i: (i, 0)),
                  pl.BlockSpec((D, D), lambda i: (0, 0))],
        out_specs=pl.BlockSpec((BLKU, D), lambda i: (i, 0)),
        out_shape=jax.ShapeDtypeStruct((P, D), jnp.float32),
    )(x, Pm)


def _consts():
    h = jnp.arange(H)
    j = jnp.arange(G)
    M48 = (j[:, None] // 3 == h[None, :]).astype(jnp.float32)
    M48 = jnp.pad(M48, ((0, GP - G), (0, 0)))
    permc = jnp.array([1, 2, 0])
    c3 = jnp.arange(3)
    T3 = (c3[:, None] == permc[j % 3][None, :]).astype(jnp.float32)
    T3 = jnp.pad(T3, ((0, 0), (0, GP - G)))
    j384 = jnp.arange(384)
    S3 = (j384[:, None] // 3 == jnp.arange(128)[None, :]).astype(jnp.float32)
    c = jnp.arange(D)
    perm_hat = (c % 16) * HD + c // 16
    m2 = c // 32
    r2 = c % 32
    perm_hat2 = (r2 // 2) * HD + 2 * m2 + (r2 % 2)
    d = jnp.arange(D)
    hatpos = (d % HD) * H + d // HD
    Pm = (hatpos[:, None] == jnp.arange(D)[None, :]).astype(jnp.float32).T
    return M48, T3, S3, perm_hat, perm_hat2, Pm


@jax.jit
def kernel(query, key, value, index_pair, query_batch_cnt, key_batch_cnt,
           index_pair_batch, relative_atten_weights, rpe_distance,
           Wq, bq, Wk, bk, Wv, bv, Wg1, bg1, Wg2, bg2):
    M48, T3, S3, perm_hat, perm_hat2, Pm = _consts()
    Wqh = Wq[:, perm_hat]
    bqh = bq[perm_hat]
    pe = perm_hat2[0::2]
    po = perm_hat2[1::2]
    Wke, Wko = Wk[:, pe], Wk[:, po]
    bke, bko = bk[pe], bk[po]
    Wve, Wvo = Wv[:, pe], Wv[:, po]
    bve, bvo = bv[pe], bv[po]
    Wg1pad = jnp.pad(Wg1, ((0, 0), (0, GP - G)))
    Wg1h = Wg1pad[perm_hat, :]
    Wg1e = Wg1pad[pe, :]
    Wg1o = Wg1pad[po, :]
    bg1p = jnp.pad(bg1, (0, GP - G))
    Wg2p = jnp.pad(Wg2, ((0, GP - G), (0, GP - G)))
    bg2p = jnp.pad(bg2, (0, GP - G))

    q, kp, vp, a, bqv = _stage1(query, key, value, Wqh, bqh, Wke, bke,
                                Wko, bko, Wve, bve, Wvo, bvo,
                                Wg1h, Wg1e, Wg1o, bg1p)
    idx_flat = index_pair.reshape(R)
    s_pack = _stage_s(rpe_distance.reshape(SROWS, 384), S3)
    s_flat = s_pack.reshape(R, 1)
    outs = []
    for off in (0, PH):
        dots_h, ag_h = _sc_pass1(kp, a, q, idx_flat, off)
        att_h = _stage_mid(dots_h, bqv, ag_h, rpe_distance, s_flat,
                           relative_atten_weights, Wg2p, bg2p, M48, T3, off)
        outs.append(_sc_pass2(vp, att_h, idx_flat, off))
    out_hat = jnp.concatenate(outs, axis=0)
    return _stage_unperm(out_hat, Pm)


# speedup vs baseline: 1.0818x; 1.0066x over previous
import dataclasses
import functools

import jax
import jax.numpy as jnp
from jax import lax
from jax.experimental import pallas as pl
from jax.experimental.pallas import tpu as pltpu
from jax.experimental.pallas import tpu_sc as plsc

P, NBR, D, H = 8192, 16, 512, 16
HD = D // H
G = H * 3
GP = 128
R = P * NBR

_SC_CP = pltpu.CompilerParams()
if "needs_layout_passes" in pltpu.CompilerParams.__dataclass_fields__:
    _SC_CP = dataclasses.replace(_SC_CP, needs_layout_passes=False)
NH = NBR * H
NV = D // 16


BLK1 = 256


def _bits16(x):
    b = jax.lax.bitcast_convert_type(x.astype(jnp.bfloat16), jnp.uint16)
    return b.astype(jnp.uint32)


def _proj_kernel(xq_ref, xk_ref, xv_ref, wq_ref, bq_ref, wke_ref, bke_ref,
                 wko_ref, bko_ref, wve_ref, bve_ref, wvo_ref, bvo_ref,
                 wg1q_ref, wg1e_ref, wg1o_ref, bg1_ref,
                 q_ref, k_ref, v_ref, a_ref, bqo_ref):
    f32 = jnp.float32
    q = jnp.dot(xq_ref[...], wq_ref[...], preferred_element_type=f32) + bq_ref[...]
    ke = jnp.dot(xk_ref[...], wke_ref[...], preferred_element_type=f32) + bke_ref[...]
    ko = jnp.dot(xk_ref[...], wko_ref[...], preferred_element_type=f32) + bko_ref[...]
    ve = jnp.dot(xv_ref[...], wve_ref[...], preferred_element_type=f32) + bve_ref[...]
    vo = jnp.dot(xv_ref[...], wvo_ref[...], preferred_element_type=f32) + bvo_ref[...]
    q_ref[...] = q
    k_ref[...] = jax.lax.bitcast_convert_type(
        _bits16(ke) | (_bits16(ko) << 16), jnp.int32)
    v_ref[...] = jax.lax.bitcast_convert_type(
        _bits16(ve) | (_bits16(vo) << 16), jnp.int32)
    a_ref[...] = (jnp.dot(ke, wg1e_ref[...], preferred_element_type=f32)
                  + jnp.dot(ko, wg1o_ref[...], preferred_element_type=f32))
    bqo_ref[...] = jnp.dot(q, wg1q_ref[...], preferred_element_type=f32) - bg1_ref[...]


def _stage1(query, key, value, Wqh, bqh, Wke, bke, Wko, bko, Wve, bve,
            Wvo, bvo, Wg1q, Wg1e, Wg1o, bg1):
    n_blk = P // BLK1
    DH = D // 2
    row_spec = pl.BlockSpec((BLK1, D), lambda i: (i, 0))
    w_spec = pl.BlockSpec((D, D), lambda i: (0, 0))
    b_spec = pl.BlockSpec((1, D), lambda i: (0, 0))
    wh_spec = pl.BlockSpec((D, DH), lambda i: (0, 0))
    bh_spec = pl.BlockSpec((1, DH), lambda i: (0, 0))
    g_spec = pl.BlockSpec((D, GP), lambda i: (0, 0))
    gh_spec = pl.BlockSpec((DH, GP), lambda i: (0, 0))
    gb_spec = pl.BlockSpec((1, GP), lambda i: (0, 0))
    out_row = pl.BlockSpec((BLK1, D), lambda i: (i, 0))
    out_half = pl.BlockSpec((BLK1, DH), lambda i: (i, 0))
    out_g = pl.BlockSpec((BLK1, GP), lambda i: (i, 0))
    return pl.pallas_call(
        _proj_kernel,
        grid=(n_blk,),
        in_specs=[row_spec, row_spec, row_spec, w_spec, b_spec,
                  wh_spec, bh_spec, wh_spec, bh_spec,
                  wh_spec, bh_spec, wh_spec, bh_spec,
                  g_spec, gh_spec, gh_spec, gb_spec],
        out_specs=[out_row, out_half, out_half, out_g, out_g],
        out_shape=[
            jax.ShapeDtypeStruct((P, D), jnp.float32),
            jax.ShapeDtypeStruct((P, DH), jnp.int32),
            jax.ShapeDtypeStruct((P, DH), jnp.int32),
            jax.ShapeDtypeStruct((P, GP), jnp.float32),
            jax.ShapeDtypeStruct((P, GP), jnp.float32),
        ],
    )(query, key, value, Wqh, bqh.reshape(1, D), Wke, bke.reshape(1, DH),
      Wko, bko.reshape(1, DH), Wve, bve.reshape(1, DH), Wvo,
      bvo.reshape(1, DH), Wg1q, Wg1e, Wg1o, bg1.reshape(1, GP))


NW = 32
PH = P // 2
QW = PH // NW
QB = 4
NCH = QW // QB


def _make_sc_dots(off):
  def _sc_dots(k_hbm, a_hbm, q_hbm, idx_hbm, dots_hbm, ag_hbm,
               idxall, kbuf0, kbuf1, abuf0, abuf1, qbuf0, qbuf1, dotbuf,
               semk0, semk1, sema0, sema1, semq0, semq1):
    wid = lax.axis_index("s") * 2 + lax.axis_index("c")
    q0l = wid * QW
    q0 = off + q0l
    kbuf = [kbuf0, kbuf1]
    abuf = [abuf0, abuf1]
    qbuf = [qbuf0, qbuf1]
    semk = [semk0, semk1]
    sema = [sema0, sema1]
    semq = [semq0, semq1]

    pltpu.sync_copy(idx_hbm.at[pl.ds(q0 * NBR, QW * NBR)], idxall)

    def prefetch(ch, b):
        qb = q0 + ch * QB
        idxs = idxall.at[pl.ds(ch * QB * NBR, QB * NBR)]
        pltpu.async_copy(q_hbm.at[pl.ds(qb, QB)], qbuf[b], semq[b])
        pltpu.async_copy(k_hbm.at[idxs], kbuf[b], semk[b])
        pltpu.async_copy(a_hbm.at[idxs], abuf[b], sema[b])

    def process(ch, b):
        qb = q0 + ch * QB
        pb = (q0l + ch * QB) * NBR
        idxs = idxall.at[pl.ds(ch * QB * NBR, QB * NBR)]
        pltpu.make_async_copy(k_hbm.at[idxs], kbuf[b], semk[b]).wait()
        pltpu.make_async_copy(a_hbm.at[idxs], abuf[b], sema[b]).wait()
        pltpu.make_async_copy(q_hbm.at[pl.ds(qb, QB)], qbuf[b], semq[b]).wait()

        @pl.loop(0, QB)
        def _(jq):
            qv = [qbuf[b][jq, pl.ds(c * 16, 16)] for c in range(NV)]
            for n in range(NBR):
                row = jq * NBR + n
                acc_a = None
                acc_b = None
                for m in range(NV // 2):
                    kk = plsc.bitcast(kbuf[b][row, pl.ds(m * 16, 16)],
                                      jnp.bfloat16)
                    ka, kb = plsc.unpack(
                        kk, format=plsc.PackFormat.INTERLEAVED)
                    ta = qv[2 * m] * ka
                    tb = qv[2 * m + 1] * kb
                    acc_a = ta if acc_a is None else acc_a + ta
                    acc_b = tb if acc_b is None else acc_b + tb
                dotbuf[row, pl.ds(0, 16)] = acc_a + acc_b

        pltpu.sync_copy(dotbuf, dots_hbm.at[pl.ds(pb, QB * NBR)])
        pltpu.sync_copy(abuf[b], ag_hbm.at[pl.ds(pb, QB * NBR)])

    prefetch(0, 0)

    @pl.loop(0, NCH // 2)
    def _(t):
        ch0 = t * 2
        prefetch(ch0 + 1, 1)
        process(ch0, 0)
        ch2 = ch0 + 2

        @pl.when(ch2 < NCH)
        def _():
            prefetch(ch2, 0)

        process(ch0 + 1, 1)

  return _sc_dots


def _sc_pass1(k, a, q, idx_flat, off):
    mesh = plsc.VectorSubcoreMesh(core_axis_name="c", subcore_axis_name="s")
    kern = functools.partial(
        pl.kernel,
        out_type=(
            jax.ShapeDtypeStruct((PH * NBR, H), jnp.float32),
            jax.ShapeDtypeStruct((PH * NBR, GP), jnp.float32),
        ),
        mesh=mesh,
        compiler_params=_SC_CP,
        scratch_types=[
            pltpu.VMEM((QW * NBR,), jnp.int32),
            pltpu.VMEM((QB * NBR, D // 2), jnp.int32),
            pltpu.VMEM((QB * NBR, D // 2), jnp.int32),
            pltpu.VMEM((QB * NBR, GP), jnp.float32),
            pltpu.VMEM((QB * NBR, GP), jnp.float32),
            pltpu.VMEM((QB, D), jnp.float32),
            pltpu.VMEM((QB, D), jnp.float32),
            pltpu.VMEM((QB * NBR, H), jnp.float32),
            pltpu.SemaphoreType.DMA,
            pltpu.SemaphoreType.DMA,
            pltpu.SemaphoreType.DMA,
            pltpu.SemaphoreType.DMA,
            pltpu.SemaphoreType.DMA,
            pltpu.SemaphoreType.DMA,
        ],
    )(_make_sc_dots(off))
    return kern(k, a, q, idx_flat)


SROWS = R // 128


def _scal_kernel(x_ref, s3_ref, s_ref):
    x = x_ref[...]
    n2 = jnp.dot(x * x, s3_ref[...], preferred_element_type=jnp.float32,
                 precision=lax.Precision.HIGHEST)
    ln = jnp.sqrt(n2)
    a_c, b_c = 0.001, 0.005
    ramp = 0.5 * (1.0 - jnp.cos(jnp.pi * (ln - a_c) / (b_c - a_c)))
    cut = jnp.where(ln < a_c, 0.0, jnp.where(ln > b_c, 1.0, ramp))
    s_ref[...] = jnp.sqrt(3.0) * cut / jnp.maximum(ln, 1e-12)


def _stage_s(rpe_pack, S3):
    return pl.pallas_call(
        _scal_kernel,
        grid=(1,),
        in_specs=[pl.BlockSpec((SROWS, 384), lambda i: (0, 0)),
                  pl.BlockSpec((384, 128), lambda i: (0, 0))],
        out_specs=pl.BlockSpec((SROWS, 128), lambda i: (0, 0)),
        out_shape=jax.ShapeDtypeStruct((SROWS, 128), jnp.float32),
    )(rpe_pack, S3)


BLKM = 128
RBM = BLKM * NBR


def _mid_kernel(dot_ref, bq_ref, ag_ref, rpe_ref, s_ref, rel_ref,
                wg2_ref, bg2_ref, m48_ref, t3_ref, att_ref):
    bq = bq_ref[...]
    bqexp = jnp.broadcast_to(bq[:, None, :], (BLKM, NBR, GP)).reshape(RBM, GP)
    pre = jnp.maximum(ag_ref[...] - bqexp, 0.0)
    t = jnp.dot(pre, wg2_ref[...], preferred_element_type=jnp.float32) + bg2_ref[...]

    shc = rpe_ref[...].reshape(RBM, 3) * s_ref[...]
    sht = jnp.dot(shc, t3_ref[...], preferred_element_type=jnp.float32)

    pos = jnp.dot(t * sht, m48_ref[...], preferred_element_type=jnp.float32)

    dot = dot_ref[...]
    lg = ((dot + pos).reshape(BLKM, NBR, H) + rel_ref[...]) * (
        1.0 / jnp.sqrt(jnp.float32(HD)))
    m = jnp.max(lg, axis=1, keepdims=True)
    e = jnp.exp(lg - m)
    w = e / jnp.sum(e, axis=1, keepdims=True)
    att_ref[...] = w.reshape(RBM, H)


def _stage_mid(dots, bqv, ag, rpe3, s_flat, rel3, Wg2, bg2, M48, T3, off):
    n_blk = dots.shape[0] // RBM
    sb = off // BLKM
    sp = off * NBR // RBM
    return pl.pallas_call(
        _mid_kernel,
        grid=(n_blk,),
        in_specs=[
            pl.BlockSpec((RBM, H), lambda i: (i, 0)),
            pl.BlockSpec((BLKM, GP), lambda i: (i + sb, 0)),
            pl.BlockSpec((RBM, GP), lambda i: (i, 0)),
            pl.BlockSpec((BLKM, NBR, 3), lambda i: (i + sb, 0, 0)),
            pl.BlockSpec((RBM, 1), lambda i: (i + sp, 0)),
            pl.BlockSpec((BLKM, NBR, H), lambda i: (i + sb, 0, 0)),
            pl.BlockSpec((GP, GP), lambda i: (0, 0)),
            pl.BlockSpec((1, GP), lambda i: (0, 0)),
            pl.BlockSpec((GP, H), lambda i: (0, 0)),
            pl.BlockSpec((3, GP), lambda i: (0, 0)),
        ],
        out_specs=pl.BlockSpec((RBM, H), lambda i: (i, 0)),
        out_shape=jax.ShapeDtypeStruct((dots.shape[0], H), jnp.float32),
    )(dots, bqv, ag, rpe3, s_flat, rel3, Wg2, bg2.reshape(1, GP),
      M48, T3)


def _make_sc_wsum(off):
  def _sc_wsum(v_hbm, att_hbm, idx_hbm, out_hbm,
               idxall, vbuf0, vbuf1, attbuf0, attbuf1, outbuf,
               semv0, semv1, sematt0, sematt1):
    wid = lax.axis_index("s") * 2 + lax.axis_index("c")
    q0l = wid * QW
    q0 = off + q0l
    vbuf = [vbuf0, vbuf1]
    attbuf = [attbuf0, attbuf1]
    semv = [semv0, semv1]
    sematt = [sematt0, sematt1]

    pltpu.sync_copy(idx_hbm.at[pl.ds(q0 * NBR, QW * NBR)], idxall)


    def prefetch(ch, b):
        pb = (q0l + ch * QB) * NBR
        idxs = idxall.at[pl.ds(ch * QB * NBR, QB * NBR)]
        pltpu.async_copy(att_hbm.at[pl.ds(pb, QB * NBR)], attbuf[b], sematt[b])
        pltpu.async_copy(v_hbm.at[idxs], vbuf[b], semv[b])

    def process(ch, b):
        qb = q0l + ch * QB
        pb = qb * NBR
        idxs = idxall.at[pl.ds(ch * QB * NBR, QB * NBR)]
        pltpu.make_async_copy(v_hbm.at[idxs], vbuf[b], semv[b]).wait()
        pltpu.make_async_copy(att_hbm.at[pl.ds(pb, QB * NBR)], attbuf[b],
                              sematt[b]).wait()

        @pl.loop(0, QB)
        def _(jq):
            att = [attbuf[b][jq * NBR + n, pl.ds(0, 16)] for n in range(NBR)]
            for m in range(NV // 2):
                acc_a = None
                acc_b = None
                for n in range(NBR):
                    vv = plsc.bitcast(
                        vbuf[b][jq * NBR + n, pl.ds(m * 16, 16)],
                        jnp.bfloat16)
                    va, vb = plsc.unpack(
                        vv, format=plsc.PackFormat.INTERLEAVED)
                    ta = att[n] * va
                    tb = att[n] * vb
                    acc_a = ta if acc_a is None else acc_a + ta
                    acc_b = tb if acc_b is None else acc_b + tb
                outbuf[jq, pl.ds((2 * m) * 16, 16)] = acc_a
                outbuf[jq, pl.ds((2 * m + 1) * 16, 16)] = acc_b

        pltpu.sync_copy(outbuf, out_hbm.at[pl.ds(qb, QB)])

    prefetch(0, 0)

    @pl.loop(0, NCH // 2)
    def _(t):
        ch0 = t * 2
        prefetch(ch0 + 1, 1)
        process(ch0, 0)
        ch2 = ch0 + 2

        @pl.when(ch2 < NCH)
        def _():
            prefetch(ch2, 0)

        process(ch0 + 1, 1)

  return _sc_wsum


def _sc_pass2(v, att, idx_flat, off):
    mesh = plsc.VectorSubcoreMesh(core_axis_name="c", subcore_axis_name="s")
    kern = functools.partial(
        pl.kernel,
        out_type=jax.ShapeDtypeStruct((PH, D), jnp.float32),
        mesh=mesh,
        compiler_params=_SC_CP,
        scratch_types=[
            pltpu.VMEM((QW * NBR,), jnp.int32),
            pltpu.VMEM((QB * NBR, D // 2), jnp.int32),
            pltpu.VMEM((QB * NBR, D // 2), jnp.int32),
            pltpu.VMEM((QB * NBR, H), jnp.float32),
            pltpu.VMEM((QB * NBR, H), jnp.float32),
            pltpu.VMEM((QB, D), jnp.float32),
            pltpu.SemaphoreType.DMA,
            pltpu.SemaphoreType.DMA,
            pltpu.SemaphoreType.DMA,
            pltpu.SemaphoreType.DMA,
        ],
    )(_make_sc_wsum(off))
    return kern(v, att, idx_flat)


BLKU = 512


def _unperm_kernel(x_ref, pm_ref, o_ref):
    o_ref[...] = jnp.dot(x_ref[...], pm_ref[...],
                         preferred_element_type=jnp.float32,
                         precision=lax.Precision.HIGHEST)


def _stage_unperm(x, Pm):
    return pl.pallas_call(
        _unperm_kernel,
        grid=(P // BLKU,),
        in_specs=[pl.BlockSpec((BLKU, D), lambda i: (i, 0)),
                  pl.BlockSpec((D, D), lambda i: (0, 0))],
        out_specs=pl.BlockSpec((BLKU, D), lambda i: (i, 0)),
        out_shape=jax.ShapeDtypeStruct((P, D), jnp.float32),
    )(x, Pm)


def _consts():
    h = jnp.arange(H)
    j = jnp.arange(G)
    M48 = (j[:, None] // 3 == h[None, :]).astype(jnp.float32)
    M48 = jnp.pad(M48, ((0, GP - G), (0, 0)))
    permc = jnp.array([1, 2, 0])
    c3 = jnp.arange(3)
    T3 = (c3[:, None] == permc[j % 3][None, :]).astype(jnp.float32)
    T3 = jnp.pad(T3, ((0, 0), (0, GP - G)))
    j384 = jnp.arange(384)
    S3 = (j384[:, None] // 3 == jnp.arange(128)[None, :]).astype(jnp.float32)
    c = jnp.arange(D)
    perm_hat = (c % 16) * HD + c // 16
    m2 = c // 32
    r2 = c % 32
    perm_hat2 = (r2 // 2) * HD + 2 * m2 + (r2 % 2)
    d = jnp.arange(D)
    hatpos = (d % HD) * H + d // HD
    Pm = (hatpos[:, None] == jnp.arange(D)[None, :]).astype(jnp.float32).T
    return M48, T3, S3, perm_hat, perm_hat2, Pm


@jax.jit
def kernel(query, key, value, index_pair, query_batch_cnt, key_batch_cnt,
           index_pair_batch, relative_atten_weights, rpe_distance,
           Wq, bq, Wk, bk, Wv, bv, Wg1, bg1, Wg2, bg2):
    M48, T3, S3, perm_hat, perm_hat2, Pm = _consts()
    Wqh = Wq[:, perm_hat]
    bqh = bq[perm_hat]
    pe = perm_hat2[0::2]
    po = perm_hat2[1::2]
    Wke, Wko = Wk[:, pe], Wk[:, po]
    bke, bko = bk[pe], bk[po]
    Wve, Wvo = Wv[:, pe], Wv[:, po]
    bve, bvo = bv[pe], bv[po]
    Wg1pad = jnp.pad(Wg1, ((0, 0), (0, GP - G)))
    Wg1h = Wg1pad[perm_hat, :]
    Wg1e = Wg1pad[pe, :]
    Wg1o = Wg1pad[po, :]
    bg1p = jnp.pad(bg1, (0, GP - G))
    Wg2p = jnp.pad(Wg2, ((0, GP - G), (0, GP - G)))
    bg2p = jnp.pad(bg2, (0, GP - G))

    q, kp, vp, a, bqv = _stage1(query, key, value, Wqh, bqh, Wke, bke,
                                Wko, bko, Wve, bve, Wvo, bvo,
                                Wg1h, Wg1e, Wg1o, bg1p)
    idx_flat = index_pair.reshape(R)
    s_pack = _stage_s(rpe_distance.reshape(SROWS, 384), S3)
    s_flat = s_pack.reshape(R, 1)
    outs = []
    for off in (0, PH):
        dots_h, ag_h = _sc_pass1(kp, a, q, idx_flat, off)
        att_h = _stage_mid(dots_h, bqv, ag_h, rpe_distance, s_flat,
                           relative_atten_weights, Wg2p, bg2p, M48, T3, off)
        outs.append(_sc_pass2(vp, att_h, idx_flat, off))
    out_hat = jnp.concatenate(outs, axis=0)
    return _stage_unperm(out_hat, Pm)


# speedup vs baseline: 1.1053x; 1.0217x over previous
import dataclasses
import functools

import jax
import jax.numpy as jnp
from jax import lax
from jax.experimental import pallas as pl
from jax.experimental.pallas import tpu as pltpu
from jax.experimental.pallas import tpu_sc as plsc

P, NBR, D, H = 8192, 16, 512, 16
HD = D // H
G = H * 3
GP = 128
R = P * NBR

_SC_CP = pltpu.CompilerParams()
if "needs_layout_passes" in pltpu.CompilerParams.__dataclass_fields__:
    _SC_CP = dataclasses.replace(_SC_CP, needs_layout_passes=False)
NH = NBR * H
NV = D // 16


BLK1 = 256


def _bits16(x):
    b = jax.lax.bitcast_convert_type(x.astype(jnp.bfloat16), jnp.uint16)
    return b.astype(jnp.uint32)


def _proj_kernel(xq_ref, xk_ref, xv_ref, wq_ref, bq_ref, wke_ref, bke_ref,
                 wko_ref, bko_ref, wve_ref, bve_ref, wvo_ref, bvo_ref,
                 wg1q_ref, wg1e_ref, wg1o_ref, bg1_ref,
                 q_ref, k_ref, v_ref, a_ref, bqo_ref):
    f32 = jnp.float32
    bf = jnp.bfloat16
    xq = xq_ref[...].astype(bf)
    xk = xk_ref[...].astype(bf)
    xv = xv_ref[...].astype(bf)
    q = jnp.dot(xq, wq_ref[...].astype(bf), preferred_element_type=f32) + bq_ref[...]
    ke = jnp.dot(xk, wke_ref[...].astype(bf), preferred_element_type=f32) + bke_ref[...]
    ko = jnp.dot(xk, wko_ref[...].astype(bf), preferred_element_type=f32) + bko_ref[...]
    ve = jnp.dot(xv, wve_ref[...].astype(bf), preferred_element_type=f32) + bve_ref[...]
    vo = jnp.dot(xv, wvo_ref[...].astype(bf), preferred_element_type=f32) + bvo_ref[...]
    q_ref[...] = q
    k_ref[...] = jax.lax.bitcast_convert_type(
        _bits16(ke) | (_bits16(ko) << 16), jnp.int32)
    v_ref[...] = jax.lax.bitcast_convert_type(
        _bits16(ve) | (_bits16(vo) << 16), jnp.int32)
    a_ref[...] = (jnp.dot(ke, wg1e_ref[...], preferred_element_type=f32)
                  + jnp.dot(ko, wg1o_ref[...], preferred_element_type=f32))
    bqo_ref[...] = jnp.dot(q, wg1q_ref[...], preferred_element_type=f32) - bg1_ref[...]


def _stage1(query, key, value, Wqh, bqh, Wke, bke, Wko, bko, Wve, bve,
            Wvo, bvo, Wg1q, Wg1e, Wg1o, bg1):
    n_blk = P // BLK1
    DH = D // 2
    row_spec = pl.BlockSpec((BLK1, D), lambda i: (i, 0))
    w_spec = pl.BlockSpec((D, D), lambda i: (0, 0))
    b_spec = pl.BlockSpec((1, D), lambda i: (0, 0))
    wh_spec = pl.BlockSpec((D, DH), lambda i: (0, 0))
    bh_spec = pl.BlockSpec((1, DH), lambda i: (0, 0))
    g_spec = pl.BlockSpec((D, GP), lambda i: (0, 0))
    gh_spec = pl.BlockSpec((DH, GP), lambda i: (0, 0))
    gb_spec = pl.BlockSpec((1, GP), lambda i: (0, 0))
    out_row = pl.BlockSpec((BLK1, D), lambda i: (i, 0))
    out_half = pl.BlockSpec((BLK1, DH), lambda i: (i, 0))
    out_g = pl.BlockSpec((BLK1, GP), lambda i: (i, 0))
    return pl.pallas_call(
        _proj_kernel,
        grid=(n_blk,),
        in_specs=[row_spec, row_spec, row_spec, w_spec, b_spec,
                  wh_spec, bh_spec, wh_spec, bh_spec,
                  wh_spec, bh_spec, wh_spec, bh_spec,
                  g_spec, gh_spec, gh_spec, gb_spec],
        out_specs=[out_row, out_half, out_half, out_g, out_g],
        out_shape=[
            jax.ShapeDtypeStruct((P, D), jnp.float32),
            jax.ShapeDtypeStruct((P, DH), jnp.int32),
            jax.ShapeDtypeStruct((P, DH), jnp.int32),
            jax.ShapeDtypeStruct((P, GP), jnp.float32),
            jax.ShapeDtypeStruct((P, GP), jnp.float32),
        ],
    )(query, key, value, Wqh, bqh.reshape(1, D), Wke, bke.reshape(1, DH),
      Wko, bko.reshape(1, DH), Wve, bve.reshape(1, DH), Wvo,
      bvo.reshape(1, DH), Wg1q, Wg1e, Wg1o, bg1.reshape(1, GP))


NW = 32
PH = P // 2
QW = PH // NW
QB = 4
NCH = QW // QB


def _make_sc_dots(off):
  def _sc_dots(k_hbm, a_hbm, q_hbm, idx_hbm, dots_hbm, ag_hbm,
               idxall, kbuf0, kbuf1, abuf0, abuf1, qbuf0, qbuf1, dotbuf,
               semk0, semk1, sema0, sema1, semq0, semq1):
    wid = lax.axis_index("s") * 2 + lax.axis_index("c")
    q0l = wid * QW
    q0 = off + q0l
    kbuf = [kbuf0, kbuf1]
    abuf = [abuf0, abuf1]
    qbuf = [qbuf0, qbuf1]
    semk = [semk0, semk1]
    sema = [sema0, sema1]
    semq = [semq0, semq1]

    pltpu.sync_copy(idx_hbm.at[pl.ds(q0 * NBR, QW * NBR)], idxall)

    def prefetch(ch, b):
        qb = q0 + ch * QB
        idxs = idxall.at[pl.ds(ch * QB * NBR, QB * NBR)]
        pltpu.async_copy(q_hbm.at[pl.ds(qb, QB)], qbuf[b], semq[b])
        pltpu.async_copy(k_hbm.at[idxs], kbuf[b], semk[b])
        pltpu.async_copy(a_hbm.at[idxs], abuf[b], sema[b])

    def process(ch, b):
        qb = q0 + ch * QB
        pb = (q0l + ch * QB) * NBR
        idxs = idxall.at[pl.ds(ch * QB * NBR, QB * NBR)]
        pltpu.make_async_copy(k_hbm.at[idxs], kbuf[b], semk[b]).wait()
        pltpu.make_async_copy(a_hbm.at[idxs], abuf[b], sema[b]).wait()
        pltpu.make_async_copy(q_hbm.at[pl.ds(qb, QB)], qbuf[b], semq[b]).wait()

        @pl.loop(0, QB)
        def _(jq):
            qv = [qbuf[b][jq, pl.ds(c * 16, 16)] for c in range(NV)]
            for n in range(NBR):
                row = jq * NBR + n
                acc_a = None
                acc_b = None
                for m in range(NV // 2):
                    kk = plsc.bitcast(kbuf[b][row, pl.ds(m * 16, 16)],
                                      jnp.bfloat16)
                    ka, kb = plsc.unpack(
                        kk, format=plsc.PackFormat.INTERLEAVED)
                    ta = qv[2 * m] * ka
                    tb = qv[2 * m + 1] * kb
                    acc_a = ta if acc_a is None else acc_a + ta
                    acc_b = tb if acc_b is None else acc_b + tb
                dotbuf[row, pl.ds(0, 16)] = acc_a + acc_b

        pltpu.sync_copy(dotbuf, dots_hbm.at[pl.ds(pb, QB * NBR)])
        pltpu.sync_copy(abuf[b], ag_hbm.at[pl.ds(pb, QB * NBR)])

    prefetch(0, 0)

    @pl.loop(0, NCH // 2)
    def _(t):
        ch0 = t * 2
        prefetch(ch0 + 1, 1)
        process(ch0, 0)
        ch2 = ch0 + 2

        @pl.when(ch2 < NCH)
        def _():
            prefetch(ch2, 0)

        process(ch0 + 1, 1)

  return _sc_dots


def _sc_pass1(k, a, q, idx_flat, off):
    mesh = plsc.VectorSubcoreMesh(core_axis_name="c", subcore_axis_name="s")
    kern = functools.partial(
        pl.kernel,
        out_type=(
            jax.ShapeDtypeStruct((PH * NBR, H), jnp.float32),
            jax.ShapeDtypeStruct((PH * NBR, GP), jnp.float32),
        ),
        mesh=mesh,
        compiler_params=_SC_CP,
        scratch_types=[
            pltpu.VMEM((QW * NBR,), jnp.int32),
            pltpu.VMEM((QB * NBR, D // 2), jnp.int32),
            pltpu.VMEM((QB * NBR, D // 2), jnp.int32),
            pltpu.VMEM((QB * NBR, GP), jnp.float32),
            pltpu.VMEM((QB * NBR, GP), jnp.float32),
            pltpu.VMEM((QB, D), jnp.float32),
            pltpu.VMEM((QB, D), jnp.float32),
            pltpu.VMEM((QB * NBR, H), jnp.float32),
            pltpu.SemaphoreType.DMA,
            pltpu.SemaphoreType.DMA,
            pltpu.SemaphoreType.DMA,
            pltpu.SemaphoreType.DMA,
            pltpu.SemaphoreType.DMA,
            pltpu.SemaphoreType.DMA,
        ],
    )(_make_sc_dots(off))
    return kern(k, a, q, idx_flat)


SROWS = R // 128


def _scal_kernel(x_ref, s3_ref, s_ref):
    x = x_ref[...]
    n2 = jnp.dot(x * x, s3_ref[...], preferred_element_type=jnp.float32,
                 precision=lax.Precision.HIGHEST)
    ln = jnp.sqrt(n2)
    a_c, b_c = 0.001, 0.005
    ramp = 0.5 * (1.0 - jnp.cos(jnp.pi * (ln - a_c) / (b_c - a_c)))
    cut = jnp.where(ln < a_c, 0.0, jnp.where(ln > b_c, 1.0, ramp))
    s_ref[...] = jnp.sqrt(3.0) * cut / jnp.maximum(ln, 1e-12)


def _stage_s(rpe_pack, S3):
    return pl.pallas_call(
        _scal_kernel,
        grid=(1,),
        in_specs=[pl.BlockSpec((SROWS, 384), lambda i: (0, 0)),
                  pl.BlockSpec((384, 128), lambda i: (0, 0))],
        out_specs=pl.BlockSpec((SROWS, 128), lambda i: (0, 0)),
        out_shape=jax.ShapeDtypeStruct((SROWS, 128), jnp.float32),
    )(rpe_pack, S3)


BLKM = 128
RBM = BLKM * NBR


def _mid_kernel(dot_ref, bq_ref, ag_ref, rpe_ref, s_ref, rel_ref,
                wg2_ref, bg2_ref, m48_ref, t3_ref, att_ref):
    bq = bq_ref[...]
    bqexp = jnp.broadcast_to(bq[:, None, :], (BLKM, NBR, GP)).reshape(RBM, GP)
    pre = jnp.maximum(ag_ref[...] - bqexp, 0.0)
    t = jnp.dot(pre, wg2_ref[...], preferred_element_type=jnp.float32) + bg2_ref[...]

    shc = rpe_ref[...].reshape(RBM, 3) * s_ref[...]
    sht = jnp.dot(shc, t3_ref[...], preferred_element_type=jnp.float32)

    pos = jnp.dot(t * sht, m48_ref[...], preferred_element_type=jnp.float32)

    dot = dot_ref[...]
    lg = ((dot + pos).reshape(BLKM, NBR, H) + rel_ref[...]) * (
        1.0 / jnp.sqrt(jnp.float32(HD)))
    m = jnp.max(lg, axis=1, keepdims=True)
    e = jnp.exp(lg - m)
    w = e / jnp.sum(e, axis=1, keepdims=True)
    att_ref[...] = w.reshape(RBM, H)


def _stage_mid(dots, bqv, ag, rpe3, s_flat, rel3, Wg2, bg2, M48, T3, off):
    n_blk = dots.shape[0] // RBM
    sb = off // BLKM
    sp = off * NBR // RBM
    return pl.pallas_call(
        _mid_kernel,
        grid=(n_blk,),
        in_specs=[
            pl.BlockSpec((RBM, H), lambda i: (i, 0)),
            pl.BlockSpec((BLKM, GP), lambda i: (i + sb, 0)),
            pl.BlockSpec((RBM, GP), lambda i: (i, 0)),
            pl.BlockSpec((BLKM, NBR, 3), lambda i: (i + sb, 0, 0)),
            pl.BlockSpec((RBM, 1), lambda i: (i + sp, 0)),
            pl.BlockSpec((BLKM, NBR, H), lambda i: (i + sb, 0, 0)),
            pl.BlockSpec((GP, GP), lambda i: (0, 0)),
            pl.BlockSpec((1, GP), lambda i: (0, 0)),
            pl.BlockSpec((GP, H), lambda i: (0, 0)),
            pl.BlockSpec((3, GP), lambda i: (0, 0)),
        ],
        out_specs=pl.BlockSpec((RBM, H), lambda i: (i, 0)),
        out_shape=jax.ShapeDtypeStruct((dots.shape[0], H), jnp.float32),
    )(dots, bqv, ag, rpe3, s_flat, rel3, Wg2, bg2.reshape(1, GP),
      M48, T3)


def _make_sc_wsum(off):
  def _sc_wsum(v_hbm, att_hbm, idx_hbm, out_hbm,
               idxall, vbuf0, vbuf1, attbuf0, attbuf1, outbuf,
               semv0, semv1, sematt0, sematt1):
    wid = lax.axis_index("s") * 2 + lax.axis_index("c")
    q0l = wid * QW
    q0 = off + q0l
    vbuf = [vbuf0, vbuf1]
    attbuf = [attbuf0, attbuf1]
    semv = [semv0, semv1]
    sematt = [sematt0, sematt1]

    pltpu.sync_copy(idx_hbm.at[pl.ds(q0 * NBR, QW * NBR)], idxall)


    def prefetch(ch, b):
        pb = (q0l + ch * QB) * NBR
        idxs = idxall.at[pl.ds(ch * QB * NBR, QB * NBR)]
        pltpu.async_copy(att_hbm.at[pl.ds(pb, QB * NBR)], attbuf[b], sematt[b])
        pltpu.async_copy(v_hbm.at[idxs], vbuf[b], semv[b])

    def process(ch, b):
        qb = q0l + ch * QB
        pb = qb * NBR
        idxs = idxall.at[pl.ds(ch * QB * NBR, QB * NBR)]
        pltpu.make_async_copy(v_hbm.at[idxs], vbuf[b], semv[b]).wait()
        pltpu.make_async_copy(att_hbm.at[pl.ds(pb, QB * NBR)], attbuf[b],
                              sematt[b]).wait()

        @pl.loop(0, QB)
        def _(jq):
            att = [attbuf[b][jq * NBR + n, pl.ds(0, 16)] for n in range(NBR)]
            for m in range(NV // 2):
                acc_a = None
                acc_b = None
                for n in range(NBR):
                    vv = plsc.bitcast(
                        vbuf[b][jq * NBR + n, pl.ds(m * 16, 16)],
                        jnp.bfloat16)
                    va, vb = plsc.unpack(
                        vv, format=plsc.PackFormat.INTERLEAVED)
                    ta = att[n] * va
                    tb = att[n] * vb
                    acc_a = ta if acc_a is None else acc_a + ta
                    acc_b = tb if acc_b is None else acc_b + tb
                outbuf[jq, pl.ds((2 * m) * 16, 16)] = acc_a
                outbuf[jq, pl.ds((2 * m + 1) * 16, 16)] = acc_b

        pltpu.sync_copy(outbuf, out_hbm.at[pl.ds(qb, QB)])

    prefetch(0, 0)

    @pl.loop(0, NCH // 2)
    def _(t):
        ch0 = t * 2
        prefetch(ch0 + 1, 1)
        process(ch0, 0)
        ch2 = ch0 + 2

        @pl.when(ch2 < NCH)
        def _():
            prefetch(ch2, 0)

        process(ch0 + 1, 1)

  return _sc_wsum


def _sc_pass2(v, att, idx_flat, off):
    mesh = plsc.VectorSubcoreMesh(core_axis_name="c", subcore_axis_name="s")
    kern = functools.partial(
        pl.kernel,
        out_type=jax.ShapeDtypeStruct((PH, D), jnp.float32),
        mesh=mesh,
        compiler_params=_SC_CP,
        scratch_types=[
            pltpu.VMEM((QW * NBR,), jnp.int32),
            pltpu.VMEM((QB * NBR, D // 2), jnp.int32),
            pltpu.VMEM((QB * NBR, D // 2), jnp.int32),
            pltpu.VMEM((QB * NBR, H), jnp.float32),
            pltpu.VMEM((QB * NBR, H), jnp.float32),
            pltpu.VMEM((QB, D), jnp.float32),
            pltpu.SemaphoreType.DMA,
            pltpu.SemaphoreType.DMA,
            pltpu.SemaphoreType.DMA,
            pltpu.SemaphoreType.DMA,
        ],
    )(_make_sc_wsum(off))
    return kern(v, att, idx_flat)


BLKU = 512


def _unperm_kernel(x_ref, pm_ref, o_ref):
    o_ref[...] = jnp.dot(x_ref[...], pm_ref[...],
                         preferred_element_type=jnp.float32,
                         precision=lax.Precision.HIGHEST)


def _stage_unperm(x, Pm):
    return pl.pallas_call(
        _unperm_kernel,
        grid=(x.shape[0] // BLKU,),
        in_specs=[pl.BlockSpec((BLKU, D), lambda i: (i, 0)),
                  pl.BlockSpec((D, D), lambda i: (0, 0))],
        out_specs=pl.BlockSpec((BLKU, D), lambda i: (i, 0)),
        out_shape=jax.ShapeDtypeStruct((x.shape[0], D), jnp.float32),
    )(x, Pm)


def _consts():
    h = jnp.arange(H)
    j = jnp.arange(G)
    M48 = (j[:, None] // 3 == h[None, :]).astype(jnp.float32)
    M48 = jnp.pad(M48, ((0, GP - G), (0, 0)))
    permc = jnp.array([1, 2, 0])
    c3 = jnp.arange(3)
    T3 = (c3[:, None] == permc[j % 3][None, :]).astype(jnp.float32)
    T3 = jnp.pad(T3, ((0, 0), (0, GP - G)))
    j384 = jnp.arange(384)
    S3 = (j384[:, None] // 3 == jnp.arange(128)[None, :]).astype(jnp.float32)
    c = jnp.arange(D)
    perm_hat = (c % 16) * HD + c // 16
    m2 = c // 32
    r2 = c % 32
    perm_hat2 = (r2 // 2) * HD + 2 * m2 + (r2 % 2)
    d = jnp.arange(D)
    hatpos = (d % HD) * H + d // HD
    Pm = (hatpos[:, None] == jnp.arange(D)[None, :]).astype(jnp.float32).T
    return M48, T3, S3, perm_hat, perm_hat2, Pm


@jax.jit
def kernel(query, key, value, index_pair, query_batch_cnt, key_batch_cnt,
           index_pair_batch, relative_atten_weights, rpe_distance,
           Wq, bq, Wk, bk, Wv, bv, Wg1, bg1, Wg2, bg2):
    M48, T3, S3, perm_hat, perm_hat2, Pm = _consts()
    Wqh = Wq[:, perm_hat]
    bqh = bq[perm_hat]
    pe = perm_hat2[0::2]
    po = perm_hat2[1::2]
    Wke, Wko = Wk[:, pe], Wk[:, po]
    bke, bko = bk[pe], bk[po]
    Wve, Wvo = Wv[:, pe], Wv[:, po]
    bve, bvo = bv[pe], bv[po]
    Wg1pad = jnp.pad(Wg1, ((0, 0), (0, GP - G)))
    Wg1h = Wg1pad[perm_hat, :]
    Wg1e = Wg1pad[pe, :]
    Wg1o = Wg1pad[po, :]
    bg1p = jnp.pad(bg1, (0, GP - G))
    Wg2p = jnp.pad(Wg2, ((0, GP - G), (0, GP - G)))
    bg2p = jnp.pad(bg2, (0, GP - G))

    q, kp, vp, a, bqv = _stage1(query, key, value, Wqh, bqh, Wke, bke,
                                Wko, bko, Wve, bve, Wvo, bvo,
                                Wg1h, Wg1e, Wg1o, bg1p)
    idx_flat = index_pair.reshape(R)
    s_pack = _stage_s(rpe_distance.reshape(SROWS, 384), S3)
    s_flat = s_pack.reshape(R, 1)
    outs = []
    for off in (0, PH):
        dots_h, ag_h = _sc_pass1(kp, a, q, idx_flat, off)
        att_h = _stage_mid(dots_h, bqv, ag_h, rpe_distance, s_flat,
                           relative_atten_weights, Wg2p, bg2p, M48, T3, off)
        out_h = _sc_pass2(vp, att_h, idx_flat, off)
        outs.append(_stage_unperm(out_h, Pm))
    return jnp.concatenate(outs, axis=0)
